# baseline (speedup 1.0000x reference)
import jax
import jax.numpy as jnp
from jax.experimental import pallas as pl


def kernel(x, W, gamma, beta):
    k = 20
    B, C, N = x.shape
    pts = jnp.transpose(x, (0, 2, 1))
    sq = jnp.sum(pts * pts, axis=-1)
    d2 = sq[:, :, None] + sq[:, None, :] - 2.0 * jnp.einsum('bnc,bmc->bnm', pts, pts)
    _, idx = jax.lax.top_k(-d2, k)

    W1 = W[:, :C]
    W2 = W[:, C:]
    A = jnp.einsum('oc,bcn->bon', W1 - W2, x)
    Bv = jnp.einsum('oc,bcn->bon', W2, x)

    bidx = jnp.arange(B)[:, None, None]
    g = Bv[bidx, :, idx]
    M = jnp.max(g, axis=2)
    S = jnp.sum(g, axis=2)
    SS = jnp.sum(g * g, axis=2)

    cnt = B * N * k
    sumA = jnp.sum(A, axis=(0, 2))
    sumA2 = jnp.sum(A * A, axis=(0, 2))
    crossAS = jnp.sum(A * jnp.transpose(S, (0, 2, 1)), axis=(0, 2))
    tot = k * sumA + jnp.sum(S, axis=(0, 1))
    tot2 = k * sumA2 + 2.0 * crossAS + jnp.sum(SS, axis=(0, 1))
    mean = tot / cnt
    var = tot2 / cnt - mean * mean

    vmax = A + jnp.transpose(M, (0, 2, 1))
    scale = gamma / jnp.sqrt(var + 1e-5)
    y = scale[None, :, None] * (vmax - mean[None, :, None]) + beta[None, :, None]
    return jnp.where(y >= 0, y, 0.2 * y)


# speedup vs baseline: 1.9478x; 1.9478x over previous
import functools

import jax
import jax.numpy as jnp
from jax.experimental import pallas as pl
from jax.experimental.pallas import tpu as pltpu

K = 20


def _knn_body(pts_r_ref, pts_f_ref, sq_r_ref, sq_m_ref, wd_ref, w2_ref,
              idx_ref, a_ref, bvt_ref, *, R, N):
    pts_r = pts_r_ref[0]
    pts_f = pts_f_ref[0]
    sq_r = sq_r_ref[0]
    sq_m = sq_m_ref[0]
    g = jax.lax.dot_general(pts_r, pts_f, (((1,), (1,)), ((), ())),
                            preferred_element_type=jnp.float32)
    d2 = sq_r + sq_m - 2.0 * g
    ii = jax.lax.broadcasted_iota(jnp.int32, (R, N), 1)
    cols = []
    for _ in range(K):
        m = jnp.min(d2, axis=1, keepdims=True)
        amin = jnp.min(jnp.where(d2 == m, ii, N), axis=1, keepdims=True)
        cols.append(amin)
        d2 = jnp.where(ii == amin, jnp.inf, d2)
    idx_ref[0] = jnp.concatenate(cols, axis=1)
    a_ref[0] = jax.lax.dot_general(pts_r, wd_ref[...], (((1,), (0,)), ((), ())),
                                   preferred_element_type=jnp.float32)
    bvt_ref[0] = jax.lax.dot_general(pts_r, w2_ref[...], (((1,), (0,)), ((), ())),
                                     preferred_element_type=jnp.float32)


def _knn_call(pts, sq, wd, w2t, R, interpret=False):
    B, N, C = pts.shape
    O = wd.shape[1]
    sq_r = sq[:, :, None]
    sq_m = sq[:, None, :]
    return pl.pallas_call(
        functools.partial(_knn_body, R=R, N=N),
        grid=(B, N // R),
        in_specs=[
            pl.BlockSpec((1, R, C), lambda b, r: (b, r, 0)),
            pl.BlockSpec((1, N, C), lambda b, r: (b, 0, 0)),
            pl.BlockSpec((1, R, 1), lambda b, r: (b, r, 0)),
            pl.BlockSpec((1, 1, N), lambda b, r: (b, 0, 0)),
            pl.BlockSpec((C, O), lambda b, r: (0, 0)),
            pl.BlockSpec((C, O), lambda b, r: (0, 0)),
        ],
        out_specs=[
            pl.BlockSpec((1, R, K), lambda b, r: (b, r, 0)),
            pl.BlockSpec((1, R, O), lambda b, r: (b, r, 0)),
            pl.BlockSpec((1, R, O), lambda b, r: (b, r, 0)),
        ],
        out_shape=[
            jax.ShapeDtypeStruct((B, N, K), jnp.int32),
            jax.ShapeDtypeStruct((B, N, O), jnp.float32),
            jax.ShapeDtypeStruct((B, N, O), jnp.float32),
        ],
        interpret=interpret,
    )(pts, pts, sq_r, sq_m, wd, w2t)


def kernel(x, W, gamma, beta):
    B, C, N = x.shape
    O = W.shape[0]
    pts = jnp.transpose(x, (0, 2, 1))
    sq = jnp.sum(pts * pts, axis=-1)
    W1 = W[:, :C]
    W2 = W[:, C:]
    wd = jnp.transpose(W1 - W2)
    w2t = jnp.transpose(W2)

    idx, A, Bvt = _knn_call(pts, sq, wd, w2t, R=256)

    bidx = jnp.arange(B)[:, None, None]
    g = Bvt[bidx, idx]
    M = jnp.max(g, axis=2)
    S = jnp.sum(g, axis=2)
    SS = jnp.sum(g * g, axis=2)

    cnt = B * N * K
    sumA = jnp.sum(A, axis=(0, 1))
    sumA2 = jnp.sum(A * A, axis=(0, 1))
    crossAS = jnp.sum(A * S, axis=(0, 1))
    tot = K * sumA + jnp.sum(S, axis=(0, 1))
    tot2 = K * sumA2 + 2.0 * crossAS + jnp.sum(SS, axis=(0, 1))
    mean = tot / cnt
    var = tot2 / cnt - mean * mean

    vmax = A + M
    scale = gamma / jnp.sqrt(var + 1e-5)
    y = scale[None, None, :] * (vmax - mean[None, None, :]) + beta[None, None, :]
    y = jnp.where(y >= 0, y, 0.2 * y)
    return jnp.transpose(y, (0, 2, 1))


# speedup vs baseline: 11.1375x; 5.7181x over previous
import functools

import jax
import jax.numpy as jnp
from jax import lax
from jax.experimental import pallas as pl
from jax.experimental.pallas import tpu as pltpu
from jax.experimental.pallas import tpu_sc as plsc

K = 20


def _knn_body(pts_r_ref, pts_f_ref, sq_r_ref, sq_m_ref, wd_ref, w2_ref,
              idx_ref, a_ref, bvt_ref, *, R, N):
    pts_r = pts_r_ref[0]
    pts_f = pts_f_ref[0]
    sq_r = sq_r_ref[0]
    sq_m = sq_m_ref[0]
    g = lax.dot_general(pts_r, pts_f, (((1,), (1,)), ((), ())),
                        preferred_element_type=jnp.float32)
    d2 = sq_r + sq_m - 2.0 * g
    ii = lax.broadcasted_iota(jnp.int32, (R, N), 1)
    cols = []
    for _ in range(K):
        m = jnp.min(d2, axis=1, keepdims=True)
        amin = jnp.min(jnp.where(d2 == m, ii, N), axis=1, keepdims=True)
        cols.append(amin)
        d2 = jnp.where(ii == amin, jnp.inf, d2)
    idx_ref[0] = jnp.concatenate(cols, axis=1)
    a_ref[0] = lax.dot_general(pts_r, wd_ref[...], (((1,), (0,)), ((), ())),
                               preferred_element_type=jnp.float32)
    bvt_ref[0] = lax.dot_general(pts_r, w2_ref[...], (((1,), (0,)), ((), ())),
                                 preferred_element_type=jnp.float32)


def _knn_call(pts, sq, wd, w2t, R, interpret=False):
    B, N, C = pts.shape
    O = wd.shape[1]
    sq_r = sq[:, :, None]
    sq_m = sq[:, None, :]
    return pl.pallas_call(
        functools.partial(_knn_body, R=R, N=N),
        grid=(B, N // R),
        in_specs=[
            pl.BlockSpec((1, R, C), lambda b, r: (b, r, 0)),
            pl.BlockSpec((1, N, C), lambda b, r: (b, 0, 0)),
            pl.BlockSpec((1, R, 1), lambda b, r: (b, r, 0)),
            pl.BlockSpec((1, 1, N), lambda b, r: (b, 0, 0)),
            pl.BlockSpec((C, O), lambda b, r: (0, 0)),
            pl.BlockSpec((C, O), lambda b, r: (0, 0)),
        ],
        out_specs=[
            pl.BlockSpec((1, R, K), lambda b, r: (b, r, 0)),
            pl.BlockSpec((1, R, O), lambda b, r: (b, r, 0)),
            pl.BlockSpec((1, R, O), lambda b, r: (b, r, 0)),
        ],
        out_shape=[
            jax.ShapeDtypeStruct((B, N, K), jnp.int32),
            jax.ShapeDtypeStruct((B, N, O), jnp.float32),
            jax.ShapeDtypeStruct((B, N, O), jnp.float32),
        ],
        interpret=interpret,
    )(pts, pts, sq_r, sq_m, wd, w2t)


def _sc_gather_call(bvt_flat, a_flat, gidx2):
    BN, O = bvt_flat.shape
    NW = 32
    PPW = BN // NW
    PC = 32
    EC = PC * K
    NI = EC // 128
    NCH = PPW // PC
    NQ = O // 16
    mesh = plsc.VectorSubcoreMesh(core_axis_name="c", subcore_axis_name="s")

    @functools.partial(
        pl.kernel, mesh=mesh,
        compiler_params=pltpu.CompilerParams(use_tc_tiling_on_sc=False),
        out_type=[
            jax.ShapeDtypeStruct((BN, O), jnp.float32),
            jax.ShapeDtypeStruct((NW, 8, O), jnp.float32),
        ],
        scratch_types=[
            pltpu.VMEM((NCH * NI, 128), jnp.int32),
            pltpu.VMEM((EC, O), jnp.float32),
            pltpu.VMEM((PC, O), jnp.float32),
            pltpu.VMEM((PC, O), jnp.float32),
            pltpu.VMEM((8, O), jnp.float32),
            pltpu.SemaphoreType.DMA,
        ],
    )
    def body(bvt_hbm, a_hbm, gidx_hbm, m_hbm, st_hbm,
             idx_v, rows_v, a_v, m_v, st_v, sem):
        wid = lax.axis_index("s") * 2 + lax.axis_index("c")
        zero = jnp.zeros((16,), jnp.float32)
        pltpu.sync_copy(gidx_hbm.at[wid], idx_v)

        init = tuple((zero, zero, zero, zero, zero) for _ in range(NQ))
        carry = init
        for c in range(NCH):
            p0 = pl.multiple_of(wid * PPW + c * PC, PC)
            pltpu.sync_copy(a_hbm.at[pl.ds(p0, PC)], a_v)
            cps = [pltpu.async_copy(bvt_hbm.at[idx_v.at[c * NI + i]],
                                    rows_v.at[pl.ds(i * 128, 128)], sem)
                   for i in range(NI)]
            for cp in cps:
                cp.wait()

            def point(p, pc):
                e0 = p * K
                news = []
                for q in range(NQ):
                    sl = pl.ds(q * 16, 16)
                    v = rows_v[e0, sl]
                    m = v
                    s = v
                    ss = v * v
                    for j in range(1, K):
                        v = rows_v[e0 + j, sl]
                        m = jnp.maximum(m, v)
                        s = s + v
                        ss = ss + v * v
                    m_v[p, sl] = m
                    a = a_v[p, sl]
                    sS, sSS, sX, sA, sA2 = pc[q]
                    news.append((sS + s, sSS + ss, sX + a * s, sA + a,
                                 sA2 + a * a))
                return tuple(news)

            carry = lax.fori_loop(0, PC, point, carry)
            pltpu.sync_copy(m_v, m_hbm.at[pl.ds(p0, PC)])
        for q in range(NQ):
            sl = pl.ds(q * 16, 16)
            sS, sSS, sX, sA, sA2 = carry[q]
            st_v[0, sl] = sS
            st_v[1, sl] = sSS
            st_v[2, sl] = sX
            st_v[3, sl] = sA
            st_v[4, sl] = sA2
            st_v[5, sl] = zero
            st_v[6, sl] = zero
            st_v[7, sl] = zero
        pltpu.sync_copy(st_v, st_hbm.at[wid])

    return body(bvt_flat, a_flat, gidx2)


def _combine_body(a_ref, m_ref, sc_ref, off_ref, out_ref):
    v = a_ref[0] + m_ref[0]
    y = v * sc_ref[...] + off_ref[...]
    y = jnp.where(y >= 0, y, 0.2 * y)
    out_ref[0] = jnp.transpose(y)


def _combine_call(A, M, scale, off):
    B, N, O = A.shape
    return pl.pallas_call(
        _combine_body,
        grid=(B,),
        in_specs=[
            pl.BlockSpec((1, N, O), lambda b: (b, 0, 0)),
            pl.BlockSpec((1, N, O), lambda b: (b, 0, 0)),
            pl.BlockSpec((1, O), lambda b: (0, 0)),
            pl.BlockSpec((1, O), lambda b: (0, 0)),
        ],
        out_specs=pl.BlockSpec((1, O, N), lambda b: (b, 0, 0)),
        out_shape=jax.ShapeDtypeStruct((B, O, N), jnp.float32),
    )(A, M, scale, off)


def kernel(x, W, gamma, beta):
    B, C, N = x.shape
    O = W.shape[0]
    pts = jnp.transpose(x, (0, 2, 1))
    sq = jnp.sum(pts * pts, axis=-1)
    W1 = W[:, :C]
    W2 = W[:, C:]
    wd = jnp.transpose(W1 - W2)
    w2t = jnp.transpose(W2)

    idx, A, Bvt = _knn_call(pts, sq, wd, w2t, R=256)

    bvt_flat = Bvt.reshape(B * N, O)
    a_flat = A.reshape(B * N, O)
    gidx = (idx + (jnp.arange(B, dtype=jnp.int32) * N)[:, None, None])
    gidx2 = gidx.reshape(32, -1, 128)
    M_flat, stats = _sc_gather_call(bvt_flat, a_flat, gidx2)

    st = jnp.sum(stats, axis=0)
    sumS, sumSS, crossAS, sumA, sumA2 = st[0], st[1], st[2], st[3], st[4]
    cnt = B * N * K
    tot = K * sumA + sumS
    tot2 = K * sumA2 + 2.0 * crossAS + sumSS
    mean = tot / cnt
    var = tot2 / cnt - mean * mean
    scale = gamma / jnp.sqrt(var + 1e-5)
    off = beta - scale * mean

    return _combine_call(A, M_flat.reshape(B, N, O),
                         scale.reshape(1, O), off.reshape(1, O))


# speedup vs baseline: 14.0441x; 1.2610x over previous
import functools

import jax
import jax.numpy as jnp
from jax import lax
from jax.experimental import pallas as pl
from jax.experimental.pallas import tpu as pltpu
from jax.experimental.pallas import tpu_sc as plsc

K = 20


def _knn_body(x_r_ref, x_f_ref, wd_ref, w2_ref, idx_ref, a_ref, bvt_ref,
              *, R, N):
    b = pl.program_id(0)
    x_r = x_r_ref[0]
    x_f = x_f_ref[0]
    g = lax.dot_general(x_r, x_f, (((0,), (0,)), ((), ())),
                        preferred_element_type=jnp.float32)
    sq_m = jnp.sum(x_f * x_f, axis=0, keepdims=True)
    xr2 = x_r * x_r
    ones = jnp.ones((x_r.shape[0], 1), jnp.float32)
    sq_r = lax.dot_general(xr2, ones, (((0,), (0,)), ((), ())),
                           precision=lax.Precision.HIGHEST,
                           preferred_element_type=jnp.float32)
    d2 = sq_r + sq_m - 2.0 * g
    ii = lax.broadcasted_iota(jnp.int32, (R, N), 1).astype(jnp.float32)
    big = float(N)
    cols = []
    for _ in range(K):
        m = jnp.min(d2, axis=1, keepdims=True)
        amin = jnp.min(jnp.where(d2 == m, ii, big), axis=1, keepdims=True)
        cols.append(amin)
        d2 = jnp.where(ii == amin, jnp.inf, d2)
    idx = jnp.concatenate(cols, axis=1).astype(jnp.int32)
    idx_ref[0] = idx + b * N
    a_ref[0] = lax.dot_general(x_r, wd_ref[...], (((0,), (0,)), ((), ())),
                               preferred_element_type=jnp.float32)
    bvt_ref[0] = lax.dot_general(x_r, w2_ref[...], (((0,), (0,)), ((), ())),
                                 preferred_element_type=jnp.float32)


def _knn_call(x, wd, w2t, R):
    B, C, N = x.shape
    O = wd.shape[1]
    return pl.pallas_call(
        functools.partial(_knn_body, R=R, N=N),
        grid=(B, N // R),
        in_specs=[
            pl.BlockSpec((1, C, R), lambda b, r: (b, 0, r)),
            pl.BlockSpec((1, C, N), lambda b, r: (b, 0, 0)),
            pl.BlockSpec((C, O), lambda b, r: (0, 0)),
            pl.BlockSpec((C, O), lambda b, r: (0, 0)),
        ],
        out_specs=[
            pl.BlockSpec((1, R, K), lambda b, r: (b, r, 0)),
            pl.BlockSpec((1, R, O), lambda b, r: (b, r, 0)),
            pl.BlockSpec((1, R, O), lambda b, r: (b, r, 0)),
        ],
        out_shape=[
            jax.ShapeDtypeStruct((B, N, K), jnp.int32),
            jax.ShapeDtypeStruct((B, N, O), jnp.float32),
            jax.ShapeDtypeStruct((B, N, O), jnp.float32),
        ],
    )(x, x, wd, w2t)


def _sc_gather_call(bvt_flat, a_flat, gidx2):
    BN, O = bvt_flat.shape
    NW = 32
    PPW = BN // NW
    PC = 32
    EC = PC * K
    NI = EC // 128
    NCH = PPW // PC
    NQ = O // 16
    mesh = plsc.VectorSubcoreMesh(core_axis_name="c", subcore_axis_name="s")

    @functools.partial(
        pl.kernel, mesh=mesh,
        compiler_params=pltpu.CompilerParams(use_tc_tiling_on_sc=False),
        out_type=[
            jax.ShapeDtypeStruct((BN, O), jnp.float32),
            jax.ShapeDtypeStruct((NW, 8, O), jnp.float32),
        ],
        scratch_types=[
            pltpu.VMEM((NCH * NI, 128), jnp.int32),
            pltpu.VMEM((EC, O), jnp.float32),
            pltpu.VMEM((PC, O), jnp.float32),
            pltpu.VMEM((PC, O), jnp.float32),
            pltpu.VMEM((8, O), jnp.float32),
            pltpu.SemaphoreType.DMA,
        ],
    )
    def body(bvt_hbm, a_hbm, gidx_hbm, m_hbm, st_hbm,
             idx_v, rows_v, a_v, m_v, st_v, sem):
        wid = lax.axis_index("s") * 2 + lax.axis_index("c")
        zero = jnp.zeros((16,), jnp.float32)
        pltpu.sync_copy(gidx_hbm.at[wid], idx_v)

        init = tuple((zero, zero, zero, zero, zero) for _ in range(NQ))
        carry = init
        for c in range(NCH):
            p0 = pl.multiple_of(wid * PPW + c * PC, PC)
            pltpu.sync_copy(a_hbm.at[pl.ds(p0, PC)], a_v)
            cps = [pltpu.async_copy(bvt_hbm.at[idx_v.at[c * NI + i]],
                                    rows_v.at[pl.ds(i * 128, 128)], sem)
                   for i in range(NI)]
            for cp in cps:
                cp.wait()

            def point(p, pc):
                e0 = p * K
                news = []
                for q in range(NQ):
                    sl = pl.ds(q * 16, 16)
                    v = rows_v[e0, sl]
                    m = v
                    s = v
                    ss = v * v
                    for j in range(1, K):
                        v = rows_v[e0 + j, sl]
                        m = jnp.maximum(m, v)
                        s = s + v
                        ss = ss + v * v
                    m_v[p, sl] = m
                    a = a_v[p, sl]
                    sS, sSS, sX, sA, sA2 = pc[q]
                    news.append((sS + s, sSS + ss, sX + a * s, sA + a,
                                 sA2 + a * a))
                return tuple(news)

            carry = lax.fori_loop(0, PC, point, carry)
            pltpu.sync_copy(m_v, m_hbm.at[pl.ds(p0, PC)])
        for q in range(NQ):
            sl = pl.ds(q * 16, 16)
            sS, sSS, sX, sA, sA2 = carry[q]
            st_v[0, sl] = sS
            st_v[1, sl] = sSS
            st_v[2, sl] = sX
            st_v[3, sl] = sA
            st_v[4, sl] = sA2
            st_v[5, sl] = zero
            st_v[6, sl] = zero
            st_v[7, sl] = zero
        pltpu.sync_copy(st_v, st_hbm.at[wid])

    return body(bvt_flat, a_flat, gidx2)


def _combine_body(a_ref, m_ref, st_ref, gam_ref, bet_ref, out_ref, *, B, cnt):
    st = jnp.sum(st_ref[...], axis=0, keepdims=True)
    O = gam_ref.shape[1]
    sumS = st[:, 0 * O:1 * O]
    sumSS = st[:, 1 * O:2 * O]
    crossAS = st[:, 2 * O:3 * O]
    sumA = st[:, 3 * O:4 * O]
    sumA2 = st[:, 4 * O:5 * O]
    tot = K * sumA + sumS
    tot2 = K * sumA2 + 2.0 * crossAS + sumSS
    mean = tot / cnt
    var = tot2 / cnt - mean * mean
    scale = gam_ref[...] / jnp.sqrt(var + 1e-5)
    off = bet_ref[...] - scale * mean
    for b in range(B):
        v = a_ref[b] + m_ref[b]
        y = v * scale + off
        y = jnp.where(y >= 0, y, 0.2 * y)
        out_ref[b] = jnp.transpose(y)


def _combine_call(A, M, stats_r, gamma2, beta2):
    B, N, O = A.shape
    return pl.pallas_call(
        functools.partial(_combine_body, B=B, cnt=float(B * N * K)),
        out_shape=jax.ShapeDtypeStruct((B, O, N), jnp.float32),
    )(A, M, stats_r, gamma2, beta2)


def kernel(x, W, gamma, beta):
    B, C, N = x.shape
    O = W.shape[0]
    W1 = W[:, :C]
    W2 = W[:, C:]
    wd = jnp.transpose(W1 - W2)
    w2t = jnp.transpose(W2)

    idx, A, Bvt = _knn_call(x, wd, w2t, R=256)

    bvt_flat = Bvt.reshape(B * N, O)
    a_flat = A.reshape(B * N, O)
    gidx2 = idx.reshape(32, -1, 128)
    M_flat, stats = _sc_gather_call(bvt_flat, a_flat, gidx2)

    return _combine_call(A, M_flat.reshape(B, N, O), stats.reshape(32, 8 * O),
                         gamma.reshape(1, O), beta.reshape(1, O))
